# baseline (speedup 1.0000x reference)
import functools

import jax
import jax.numpy as jnp
from jax import lax
from jax.experimental import pallas as pl
from jax.experimental.pallas import tpu as pltpu
from jax.experimental.pallas import tpu_sc as plsc

B, N, K, C = 8, 4096, 512, 64
NTOT = B * N
NB = 512
NTOK = N // 4
LANES = 16


def _tc_body(probs_ref, logits_ref, ha_ref, pc_ref):
    p = probs_ref[...]
    kio = lax.broadcasted_iota(jnp.int32, (NB, K), 1)
    pmax = jnp.max(p, axis=-1, keepdims=True)
    ha_ref[...] = jnp.min(jnp.where(p == pmax, kio, K), axis=-1)

    lt = logits_ref[...].T
    cio = lax.broadcasted_iota(jnp.int32, (C, NB), 0)
    lmax = jnp.max(lt, axis=0, keepdims=True)
    pc_ref[...] = jnp.min(jnp.where(lt == lmax, cio, C), axis=0)


def _tc_call(probs2, logits2):
    tok1 = pl.BlockSpec((NB,), lambda i: (i,))
    out1 = jax.ShapeDtypeStruct((NTOT,), jnp.int32)
    return pl.pallas_call(
        _tc_body,
        grid=(NTOT // NB,),
        in_specs=[
            pl.BlockSpec((NB, K), lambda i: (i, 0)),
            pl.BlockSpec((NB, C), lambda i: (i, 0)),
        ],
        out_specs=[tok1, tok1],
        out_shape=[out1, out1],
        compiler_params=pltpu.CompilerParams(
            dimension_semantics=("arbitrary",),
        ),
    )(probs2, logits2)


def _box_body(boxes_ref, micro_ref, x1_ref, y1_ref, x2_ref, y2_ref):
    bx = boxes_ref[...]
    micro_ref[...] = bx
    bt = bx.T
    x1_ref[...] = bt[0]
    y1_ref[...] = bt[1]
    x2_ref[...] = bt[2]
    y2_ref[...] = bt[3]


NBX = 4096


def _box_call(boxes2):
    tok1 = pl.BlockSpec((NBX,), lambda i: (i,))
    outf = jax.ShapeDtypeStruct((NTOT,), jnp.float32)
    return pl.pallas_call(
        _box_body,
        grid=(NTOT // NBX,),
        in_specs=[pl.BlockSpec((NBX, 4), lambda i: (i, 0))],
        out_specs=[pl.BlockSpec((NBX, 4), lambda i: (i, 0)),
                   tok1, tok1, tok1, tok1],
        out_shape=[jax.ShapeDtypeStruct((NTOT, 4), jnp.float32),
                   outf, outf, outf, outf],
        compiler_params=pltpu.CompilerParams(
            dimension_semantics=("arbitrary",),
        ),
    )(boxes2)


@functools.partial(
    pl.kernel,
    mesh=plsc.VectorSubcoreMesh(core_axis_name="c", subcore_axis_name="s"),
    out_type=jax.ShapeDtypeStruct((B, 4, K), jnp.float32),
    scratch_types=[
        pltpu.VMEM((NTOK,), jnp.int32),
        pltpu.VMEM((NTOK,), jnp.float32),
        pltpu.VMEM((NTOK,), jnp.float32),
        pltpu.VMEM((NTOK,), jnp.float32),
        pltpu.VMEM((NTOK,), jnp.float32),
        pltpu.VMEM((LANES * K,), jnp.float32),
        pltpu.VMEM((LANES * K,), jnp.float32),
        pltpu.VMEM((LANES * K,), jnp.float32),
        pltpu.VMEM((LANES * K,), jnp.float32),
        pltpu.VMEM((4, K), jnp.float32),
        pltpu.VMEM((4, 4, K), jnp.float32),
        pltpu.VMEM_SHARED((16, 4, K), jnp.float32),
    ],
    compiler_params=pltpu.CompilerParams(needs_layout_passes=False),
)
def _sc_scatter(ha_hbm, x1_hbm, y1_hbm, x2_hbm, y2_hbm, out_hbm,
                idxv, bx0, bx1, bx2, bx3,
                a0, a1, a2, a3, part, comb, shared):
    c = lax.axis_index("c")
    s = lax.axis_index("s")
    b = c * 4 + s // 4
    q = s % 4
    base = b * N + q * NTOK
    pltpu.sync_copy(ha_hbm.at[pl.ds(base, NTOK)], idxv)
    pltpu.sync_copy(x1_hbm.at[pl.ds(base, NTOK)], bx0)
    pltpu.sync_copy(y1_hbm.at[pl.ds(base, NTOK)], bx1)
    pltpu.sync_copy(x2_hbm.at[pl.ds(base, NTOK)], bx2)
    pltpu.sync_copy(y2_hbm.at[pl.ds(base, NTOK)], bx3)

    ones = jnp.full((LANES,), 1.0, jnp.float32)
    zeros = jnp.zeros((LANES,), jnp.float32)

    def init_body(j, carry):
        off = j * LANES
        a0[pl.ds(off, LANES)] = ones
        a1[pl.ds(off, LANES)] = ones
        a2[pl.ds(off, LANES)] = zeros
        a3[pl.ds(off, LANES)] = zeros
        return carry

    lax.fori_loop(0, K, init_body, 0)

    lane = lax.iota(jnp.int32, LANES) * K

    def tok_body(t, carry):
        off = t * LANES
        g = lane + idxv[pl.ds(off, LANES)]
        v0 = plsc.load_gather(a0, [g])
        plsc.store_scatter(a0, [g], jnp.minimum(v0, bx0[pl.ds(off, LANES)]))
        v1 = plsc.load_gather(a1, [g])
        plsc.store_scatter(a1, [g], jnp.minimum(v1, bx1[pl.ds(off, LANES)]))
        v2 = plsc.load_gather(a2, [g])
        plsc.store_scatter(a2, [g], jnp.maximum(v2, bx2[pl.ds(off, LANES)]))
        v3 = plsc.load_gather(a3, [g])
        plsc.store_scatter(a3, [g], jnp.maximum(v3, bx3[pl.ds(off, LANES)]))
        return carry

    lax.fori_loop(0, NTOK // LANES, tok_body, 0)

    def red_body(j, carry):
        off = j * LANES
        r0 = a0[pl.ds(off, LANES)]
        r1 = a1[pl.ds(off, LANES)]
        r2 = a2[pl.ds(off, LANES)]
        r3 = a3[pl.ds(off, LANES)]
        for lrep in range(1, LANES):
            r0 = jnp.minimum(r0, a0[pl.ds(lrep * K + off, LANES)])
            r1 = jnp.minimum(r1, a1[pl.ds(lrep * K + off, LANES)])
            r2 = jnp.maximum(r2, a2[pl.ds(lrep * K + off, LANES)])
            r3 = jnp.maximum(r3, a3[pl.ds(lrep * K + off, LANES)])
        part[0, pl.ds(off, LANES)] = r0
        part[1, pl.ds(off, LANES)] = r1
        part[2, pl.ds(off, LANES)] = r2
        part[3, pl.ds(off, LANES)] = r3
        return carry

    lax.fori_loop(0, K // LANES, red_body, 0)

    pltpu.sync_copy(part, shared.at[s])
    plsc.subcore_barrier()

    @pl.when(q == 0)
    def _():
        pltpu.sync_copy(shared.at[pl.ds(s, 4)], comb)

        def comb_body(j, carry):
            off = j * LANES
            for coord, op in ((0, jnp.minimum), (1, jnp.minimum),
                              (2, jnp.maximum), (3, jnp.maximum)):
                r = comb[0, coord, pl.ds(off, LANES)]
                for w in range(1, 4):
                    r = op(r, comb[w, coord, pl.ds(off, LANES)])
                part[coord, pl.ds(off, LANES)] = r
            return carry

        lax.fori_loop(0, K // LANES, comb_body, 0)
        pltpu.sync_copy(part, out_hbm.at[b])


def kernel(boxes, assign_probs, class_logits):
    micro, x1, y1, x2, y2 = _box_call(boxes.reshape(NTOT, 4))
    ha, pc = _tc_call(
        assign_probs.reshape(NTOT, K),
        class_logits.reshape(NTOT, C),
    )
    comp_t = _sc_scatter(ha, x1, y1, x2, y2)
    hard_assign = ha.reshape(B, N)
    pred_classes = pc.reshape(B, N)
    comp = comp_t.transpose(0, 2, 1)
    keep = jnp.ones((B, N), dtype=bool)
    return (hard_assign, pred_classes, micro.reshape(B, N, 4), keep, comp)


# speedup vs baseline: 1.0098x; 1.0098x over previous
import functools

import jax
import jax.numpy as jnp
from jax import lax
from jax.experimental import pallas as pl
from jax.experimental.pallas import tpu as pltpu
from jax.experimental.pallas import tpu_sc as plsc

B, N, K, C = 8, 4096, 512, 64
NTOT = B * N
NB = 512
NTOK = N // 4
LANES = 16


def _tc_body(probs_ref, logits_ref, ha_ref, pc_ref):
    p = probs_ref[...]
    kio = lax.broadcasted_iota(jnp.int32, (NB, K), 1)
    pmax = jnp.max(p, axis=-1, keepdims=True)
    ha_ref[...] = jnp.min(jnp.where(p == pmax, kio, K), axis=-1)

    lt = logits_ref[...].T
    cio = lax.broadcasted_iota(jnp.int32, (C, NB), 0)
    lmax = jnp.max(lt, axis=0, keepdims=True)
    pc_ref[...] = jnp.min(jnp.where(lt == lmax, cio, C), axis=0)


def _tc_call(probs2, logits2):
    tok1 = pl.BlockSpec((NB,), lambda i: (i,))
    out1 = jax.ShapeDtypeStruct((NTOT,), jnp.int32)
    return pl.pallas_call(
        _tc_body,
        grid=(NTOT // NB,),
        in_specs=[
            pl.BlockSpec((NB, K), lambda i: (i, 0)),
            pl.BlockSpec((NB, C), lambda i: (i, 0)),
        ],
        out_specs=[tok1, tok1],
        out_shape=[out1, out1],
        compiler_params=pltpu.CompilerParams(
            dimension_semantics=("arbitrary",),
        ),
    )(probs2, logits2)


def _box_body(boxes_ref, micro_ref, x1_ref, y1_ref, x2_ref, y2_ref):
    bx = boxes_ref[0]
    micro_ref[0] = bx
    bt = bx.T
    x1_ref[...] = bt[0]
    y1_ref[...] = bt[1]
    x2_ref[...] = bt[2]
    y2_ref[...] = bt[3]


def _box_call(boxes):
    tok1 = pl.BlockSpec((N,), lambda b: (b,))
    outf = jax.ShapeDtypeStruct((NTOT,), jnp.float32)
    return pl.pallas_call(
        _box_body,
        grid=(B,),
        in_specs=[pl.BlockSpec((1, N, 4), lambda b: (b, 0, 0))],
        out_specs=[pl.BlockSpec((1, N, 4), lambda b: (b, 0, 0)),
                   tok1, tok1, tok1, tok1],
        out_shape=[jax.ShapeDtypeStruct((B, N, 4), jnp.float32),
                   outf, outf, outf, outf],
        compiler_params=pltpu.CompilerParams(
            dimension_semantics=("arbitrary",),
        ),
    )(boxes)


@functools.partial(
    pl.kernel,
    mesh=plsc.VectorSubcoreMesh(core_axis_name="c", subcore_axis_name="s"),
    out_type=jax.ShapeDtypeStruct((B, 4, K), jnp.float32),
    scratch_types=[
        pltpu.VMEM((NTOK,), jnp.int32),
        pltpu.VMEM((NTOK,), jnp.float32),
        pltpu.VMEM((NTOK,), jnp.float32),
        pltpu.VMEM((NTOK,), jnp.float32),
        pltpu.VMEM((NTOK,), jnp.float32),
        pltpu.VMEM((LANES * K,), jnp.float32),
        pltpu.VMEM((LANES * K,), jnp.float32),
        pltpu.VMEM((LANES * K,), jnp.float32),
        pltpu.VMEM((LANES * K,), jnp.float32),
        pltpu.VMEM((4, K), jnp.float32),
        pltpu.VMEM((4, 4, K), jnp.float32),
        pltpu.VMEM_SHARED((16, 4, K), jnp.float32),
    ],
    compiler_params=pltpu.CompilerParams(needs_layout_passes=False),
)
def _sc_scatter(ha_hbm, x1_hbm, y1_hbm, x2_hbm, y2_hbm, out_hbm,
                idxv, bx0, bx1, bx2, bx3,
                a0, a1, a2, a3, part, comb, shared):
    c = lax.axis_index("c")
    s = lax.axis_index("s")
    b = c * 4 + s // 4
    q = s % 4
    base = b * N + q * NTOK
    pltpu.sync_copy(ha_hbm.at[pl.ds(base, NTOK)], idxv)
    pltpu.sync_copy(x1_hbm.at[pl.ds(base, NTOK)], bx0)
    pltpu.sync_copy(y1_hbm.at[pl.ds(base, NTOK)], bx1)
    pltpu.sync_copy(x2_hbm.at[pl.ds(base, NTOK)], bx2)
    pltpu.sync_copy(y2_hbm.at[pl.ds(base, NTOK)], bx3)

    ones = jnp.full((LANES,), 1.0, jnp.float32)
    zeros = jnp.zeros((LANES,), jnp.float32)

    def init_body(j, carry):
        off = j * LANES
        a0[pl.ds(off, LANES)] = ones
        a1[pl.ds(off, LANES)] = ones
        a2[pl.ds(off, LANES)] = zeros
        a3[pl.ds(off, LANES)] = zeros
        return carry

    lax.fori_loop(0, K, init_body, 0)

    lane = lax.iota(jnp.int32, LANES) * K

    def tok_body(t, carry):
        off = t * LANES
        g = lane + idxv[pl.ds(off, LANES)]
        v0 = plsc.load_gather(a0, [g])
        plsc.store_scatter(a0, [g], jnp.minimum(v0, bx0[pl.ds(off, LANES)]))
        v1 = plsc.load_gather(a1, [g])
        plsc.store_scatter(a1, [g], jnp.minimum(v1, bx1[pl.ds(off, LANES)]))
        v2 = plsc.load_gather(a2, [g])
        plsc.store_scatter(a2, [g], jnp.maximum(v2, bx2[pl.ds(off, LANES)]))
        v3 = plsc.load_gather(a3, [g])
        plsc.store_scatter(a3, [g], jnp.maximum(v3, bx3[pl.ds(off, LANES)]))
        return carry

    lax.fori_loop(0, NTOK // LANES, tok_body, 0)

    def red_body(j, carry):
        off = j * LANES
        r0 = a0[pl.ds(off, LANES)]
        r1 = a1[pl.ds(off, LANES)]
        r2 = a2[pl.ds(off, LANES)]
        r3 = a3[pl.ds(off, LANES)]
        for lrep in range(1, LANES):
            r0 = jnp.minimum(r0, a0[pl.ds(lrep * K + off, LANES)])
            r1 = jnp.minimum(r1, a1[pl.ds(lrep * K + off, LANES)])
            r2 = jnp.maximum(r2, a2[pl.ds(lrep * K + off, LANES)])
            r3 = jnp.maximum(r3, a3[pl.ds(lrep * K + off, LANES)])
        part[0, pl.ds(off, LANES)] = r0
        part[1, pl.ds(off, LANES)] = r1
        part[2, pl.ds(off, LANES)] = r2
        part[3, pl.ds(off, LANES)] = r3
        return carry

    lax.fori_loop(0, K // LANES, red_body, 0)

    pltpu.sync_copy(part, shared.at[s])
    plsc.subcore_barrier()

    @pl.when(q == 0)
    def _():
        pltpu.sync_copy(shared.at[pl.ds(s, 4)], comb)

        def comb_body(j, carry):
            off = j * LANES
            for coord, op in ((0, jnp.minimum), (1, jnp.minimum),
                              (2, jnp.maximum), (3, jnp.maximum)):
                r = comb[0, coord, pl.ds(off, LANES)]
                for w in range(1, 4):
                    r = op(r, comb[w, coord, pl.ds(off, LANES)])
                part[coord, pl.ds(off, LANES)] = r
            return carry

        lax.fori_loop(0, K // LANES, comb_body, 0)
        pltpu.sync_copy(part, out_hbm.at[b])


def kernel(boxes, assign_probs, class_logits):
    micro, x1, y1, x2, y2 = _box_call(boxes)
    ha, pc = _tc_call(
        assign_probs.reshape(NTOT, K),
        class_logits.reshape(NTOT, C),
    )
    comp_t = _sc_scatter(ha, x1, y1, x2, y2)
    hard_assign = ha.reshape(B, N)
    pred_classes = pc.reshape(B, N)
    comp = comp_t.transpose(0, 2, 1)
    keep = jnp.ones((B, N), dtype=bool)
    return (hard_assign, pred_classes, micro, keep, comp)


# speedup vs baseline: 1.2284x; 1.2165x over previous
import functools

import jax
import jax.numpy as jnp
from jax import lax
from jax.experimental import pallas as pl
from jax.experimental.pallas import tpu as pltpu
from jax.experimental.pallas import tpu_sc as plsc

B, N, K, C = 8, 4096, 512, 64
NTOT = B * N
NB = 1024
NTOK = N // 4
LANES = 16


def _tc_body(probs_ref, logits_ref, ha_ref, pc_ref):
    p = probs_ref[...]
    kio = lax.broadcasted_iota(jnp.int32, (NB, K), 1)
    pmax = jnp.max(p, axis=-1, keepdims=True)
    ha_ref[...] = jnp.min(jnp.where(p == pmax, kio, K), axis=-1)

    lt = logits_ref[...].T
    cio = lax.broadcasted_iota(jnp.int32, (C, NB), 0)
    lmax = jnp.max(lt, axis=0, keepdims=True)
    pc_ref[...] = jnp.min(jnp.where(lt == lmax, cio, C), axis=0)


def _tc_call(probs2, logits2):
    tok1 = pl.BlockSpec((NB,), lambda i: (i,))
    out1 = jax.ShapeDtypeStruct((NTOT,), jnp.int32)
    return pl.pallas_call(
        _tc_body,
        grid=(NTOT // NB,),
        in_specs=[
            pl.BlockSpec((NB, K), lambda i: (i, 0)),
            pl.BlockSpec((NB, C), lambda i: (i, 0)),
        ],
        out_specs=[tok1, tok1],
        out_shape=[out1, out1],
        compiler_params=pltpu.CompilerParams(
            dimension_semantics=("arbitrary",),
        ),
    )(probs2, logits2)


def _box_body(boxes_ref, x1_ref, y1_ref, x2_ref, y2_ref):
    bx = boxes_ref[0]
    bt = bx.T
    x1_ref[...] = bt[0]
    y1_ref[...] = bt[1]
    x2_ref[...] = bt[2]
    y2_ref[...] = bt[3]


def _box_call(boxes):
    tok1 = pl.BlockSpec((N,), lambda b: (b,))
    outf = jax.ShapeDtypeStruct((NTOT,), jnp.float32)
    return pl.pallas_call(
        _box_body,
        grid=(B,),
        in_specs=[pl.BlockSpec((1, N, 4), lambda b: (b, 0, 0))],
        out_specs=[tok1, tok1, tok1, tok1],
        out_shape=[outf, outf, outf, outf],
        compiler_params=pltpu.CompilerParams(
            dimension_semantics=("arbitrary",),
        ),
    )(boxes)


@functools.partial(
    pl.kernel,
    mesh=plsc.VectorSubcoreMesh(core_axis_name="c", subcore_axis_name="s"),
    out_type=jax.ShapeDtypeStruct((B, 4, K), jnp.float32),
    scratch_types=[
        pltpu.VMEM((NTOK,), jnp.int32),
        pltpu.VMEM((NTOK,), jnp.float32),
        pltpu.VMEM((NTOK,), jnp.float32),
        pltpu.VMEM((NTOK,), jnp.float32),
        pltpu.VMEM((NTOK,), jnp.float32),
        pltpu.VMEM((LANES * K,), jnp.float32),
        pltpu.VMEM((LANES * K,), jnp.float32),
        pltpu.VMEM((LANES * K,), jnp.float32),
        pltpu.VMEM((LANES * K,), jnp.float32),
        pltpu.VMEM((4, K), jnp.float32),
        pltpu.VMEM((4, 4, K), jnp.float32),
        pltpu.VMEM_SHARED((16, 4, K), jnp.float32),
    ],
    compiler_params=pltpu.CompilerParams(needs_layout_passes=False),
)
def _sc_scatter(ha_hbm, x1_hbm, y1_hbm, x2_hbm, y2_hbm, out_hbm,
                idxv, bx0, bx1, bx2, bx3,
                a0, a1, a2, a3, part, comb, shared):
    c = lax.axis_index("c")
    s = lax.axis_index("s")
    b = c * 4 + s // 4
    q = s % 4
    base = b * N + q * NTOK
    pltpu.sync_copy(ha_hbm.at[pl.ds(base, NTOK)], idxv)
    pltpu.sync_copy(x1_hbm.at[pl.ds(base, NTOK)], bx0)
    pltpu.sync_copy(y1_hbm.at[pl.ds(base, NTOK)], bx1)
    pltpu.sync_copy(x2_hbm.at[pl.ds(base, NTOK)], bx2)
    pltpu.sync_copy(y2_hbm.at[pl.ds(base, NTOK)], bx3)

    ones = jnp.full((LANES,), 1.0, jnp.float32)
    zeros = jnp.zeros((LANES,), jnp.float32)

    def init_body(j, carry):
        off = j * LANES
        a0[pl.ds(off, LANES)] = ones
        a1[pl.ds(off, LANES)] = ones
        a2[pl.ds(off, LANES)] = zeros
        a3[pl.ds(off, LANES)] = zeros
        return carry

    lax.fori_loop(0, K, init_body, 0)

    lane = lax.iota(jnp.int32, LANES) * K

    def tok_body(t, carry):
        off = t * LANES
        g = lane + idxv[pl.ds(off, LANES)]
        v0 = plsc.load_gather(a0, [g])
        plsc.store_scatter(a0, [g], jnp.minimum(v0, bx0[pl.ds(off, LANES)]))
        v1 = plsc.load_gather(a1, [g])
        plsc.store_scatter(a1, [g], jnp.minimum(v1, bx1[pl.ds(off, LANES)]))
        v2 = plsc.load_gather(a2, [g])
        plsc.store_scatter(a2, [g], jnp.maximum(v2, bx2[pl.ds(off, LANES)]))
        v3 = plsc.load_gather(a3, [g])
        plsc.store_scatter(a3, [g], jnp.maximum(v3, bx3[pl.ds(off, LANES)]))
        return carry

    lax.fori_loop(0, NTOK // LANES, tok_body, 0)

    def red_body(j, carry):
        off = j * LANES
        r0 = a0[pl.ds(off, LANES)]
        r1 = a1[pl.ds(off, LANES)]
        r2 = a2[pl.ds(off, LANES)]
        r3 = a3[pl.ds(off, LANES)]
        for lrep in range(1, LANES):
            r0 = jnp.minimum(r0, a0[pl.ds(lrep * K + off, LANES)])
            r1 = jnp.minimum(r1, a1[pl.ds(lrep * K + off, LANES)])
            r2 = jnp.maximum(r2, a2[pl.ds(lrep * K + off, LANES)])
            r3 = jnp.maximum(r3, a3[pl.ds(lrep * K + off, LANES)])
        part[0, pl.ds(off, LANES)] = r0
        part[1, pl.ds(off, LANES)] = r1
        part[2, pl.ds(off, LANES)] = r2
        part[3, pl.ds(off, LANES)] = r3
        return carry

    lax.fori_loop(0, K // LANES, red_body, 0)

    pltpu.sync_copy(part, shared.at[s])
    plsc.subcore_barrier()

    @pl.when(q == 0)
    def _():
        pltpu.sync_copy(shared.at[pl.ds(s, 4)], comb)

        def comb_body(j, carry):
            off = j * LANES
            for coord, op in ((0, jnp.minimum), (1, jnp.minimum),
                              (2, jnp.maximum), (3, jnp.maximum)):
                r = comb[0, coord, pl.ds(off, LANES)]
                for w in range(1, 4):
                    r = op(r, comb[w, coord, pl.ds(off, LANES)])
                part[coord, pl.ds(off, LANES)] = r
            return carry

        lax.fori_loop(0, K // LANES, comb_body, 0)
        pltpu.sync_copy(part, out_hbm.at[b])


def kernel(boxes, assign_probs, class_logits):
    x1, y1, x2, y2 = _box_call(boxes)
    ha, pc = _tc_call(
        assign_probs.reshape(NTOT, K),
        class_logits.reshape(NTOT, C),
    )
    comp_t = _sc_scatter(ha, x1, y1, x2, y2)
    hard_assign = ha.reshape(B, N)
    pred_classes = pc.reshape(B, N)
    comp = comp_t.transpose(0, 2, 1)
    keep = jnp.ones((B, N), dtype=bool)
    return (hard_assign, pred_classes, boxes, keep, comp)


# speedup vs baseline: 1.3336x; 1.0856x over previous
import functools

import jax
import jax.numpy as jnp
from jax import lax
from jax.experimental import pallas as pl
from jax.experimental.pallas import tpu as pltpu
from jax.experimental.pallas import tpu_sc as plsc

B, N, K, C = 8, 4096, 512, 64
NTOT = B * N
NB = 2048
NTOK = N // 4
LANES = 16


def _tc_body(probs_ref, logits_ref, ha_ref, pc_ref):
    p = probs_ref[...]
    kio = lax.broadcasted_iota(jnp.int32, (NB, K), 1)
    pmax = jnp.max(p, axis=-1, keepdims=True)
    ha_ref[...] = jnp.min(jnp.where(p == pmax, kio, K), axis=-1)

    lt = logits_ref[...].T
    cio = lax.broadcasted_iota(jnp.int32, (C, NB), 0)
    lmax = jnp.max(lt, axis=0, keepdims=True)
    pc_ref[...] = jnp.min(jnp.where(lt == lmax, cio, C), axis=0)


def _tc_call(probs2, logits2):
    tok1 = pl.BlockSpec((NB,), lambda i: (i,))
    out1 = jax.ShapeDtypeStruct((NTOT,), jnp.int32)
    return pl.pallas_call(
        _tc_body,
        grid=(NTOT // NB,),
        in_specs=[
            pl.BlockSpec((NB, K), lambda i: (i, 0)),
            pl.BlockSpec((NB, C), lambda i: (i, 0)),
        ],
        out_specs=[tok1, tok1],
        out_shape=[out1, out1],
        compiler_params=pltpu.CompilerParams(
            dimension_semantics=("arbitrary",),
        ),
    )(probs2, logits2)


def _box_body(boxes_ref, x1_ref, y1_ref, x2_ref, y2_ref):
    bx = boxes_ref[0]
    bt = bx.T
    x1_ref[...] = bt[0]
    y1_ref[...] = bt[1]
    x2_ref[...] = bt[2]
    y2_ref[...] = bt[3]


def _box_call(boxes):
    tok1 = pl.BlockSpec((N,), lambda b: (b,))
    outf = jax.ShapeDtypeStruct((NTOT,), jnp.float32)
    return pl.pallas_call(
        _box_body,
        grid=(B,),
        in_specs=[pl.BlockSpec((1, N, 4), lambda b: (b, 0, 0))],
        out_specs=[tok1, tok1, tok1, tok1],
        out_shape=[outf, outf, outf, outf],
        compiler_params=pltpu.CompilerParams(
            dimension_semantics=("arbitrary",),
        ),
    )(boxes)


@functools.partial(
    pl.kernel,
    mesh=plsc.VectorSubcoreMesh(core_axis_name="c", subcore_axis_name="s"),
    out_type=jax.ShapeDtypeStruct((B, 4, K), jnp.float32),
    scratch_types=[
        pltpu.VMEM((NTOK,), jnp.int32),
        pltpu.VMEM((NTOK,), jnp.float32),
        pltpu.VMEM((NTOK,), jnp.float32),
        pltpu.VMEM((NTOK,), jnp.float32),
        pltpu.VMEM((NTOK,), jnp.float32),
        pltpu.VMEM((LANES * K,), jnp.float32),
        pltpu.VMEM((LANES * K,), jnp.float32),
        pltpu.VMEM((LANES * K,), jnp.float32),
        pltpu.VMEM((LANES * K,), jnp.float32),
        pltpu.VMEM((4, K), jnp.float32),
        pltpu.VMEM((4, 4, K), jnp.float32),
        pltpu.VMEM_SHARED((16, 4, K), jnp.float32),
    ],
    compiler_params=pltpu.CompilerParams(needs_layout_passes=False),
)
def _sc_scatter(ha_hbm, x1_hbm, y1_hbm, x2_hbm, y2_hbm, out_hbm,
                idxv, bx0, bx1, bx2, bx3,
                a0, a1, a2, a3, part, comb, shared):
    c = lax.axis_index("c")
    s = lax.axis_index("s")
    b = c * 4 + s // 4
    q = s % 4
    base = b * N + q * NTOK
    pltpu.sync_copy(ha_hbm.at[pl.ds(base, NTOK)], idxv)
    pltpu.sync_copy(x1_hbm.at[pl.ds(base, NTOK)], bx0)
    pltpu.sync_copy(y1_hbm.at[pl.ds(base, NTOK)], bx1)
    pltpu.sync_copy(x2_hbm.at[pl.ds(base, NTOK)], bx2)
    pltpu.sync_copy(y2_hbm.at[pl.ds(base, NTOK)], bx3)

    ones = jnp.full((LANES,), 1.0, jnp.float32)
    zeros = jnp.zeros((LANES,), jnp.float32)

    def init_body(j, carry):
        off = j * LANES
        a0[pl.ds(off, LANES)] = ones
        a1[pl.ds(off, LANES)] = ones
        a2[pl.ds(off, LANES)] = zeros
        a3[pl.ds(off, LANES)] = zeros
        return carry

    lax.fori_loop(0, K, init_body, 0)

    lane = lax.iota(jnp.int32, LANES) * K

    def tok_body(t, carry):
        off = t * LANES
        g = lane + idxv[pl.ds(off, LANES)]
        v0 = plsc.load_gather(a0, [g])
        plsc.store_scatter(a0, [g], jnp.minimum(v0, bx0[pl.ds(off, LANES)]))
        v1 = plsc.load_gather(a1, [g])
        plsc.store_scatter(a1, [g], jnp.minimum(v1, bx1[pl.ds(off, LANES)]))
        v2 = plsc.load_gather(a2, [g])
        plsc.store_scatter(a2, [g], jnp.maximum(v2, bx2[pl.ds(off, LANES)]))
        v3 = plsc.load_gather(a3, [g])
        plsc.store_scatter(a3, [g], jnp.maximum(v3, bx3[pl.ds(off, LANES)]))
        return carry

    lax.fori_loop(0, NTOK // LANES, tok_body, 0)

    def red_body(j, carry):
        off = j * LANES
        r0 = a0[pl.ds(off, LANES)]
        r1 = a1[pl.ds(off, LANES)]
        r2 = a2[pl.ds(off, LANES)]
        r3 = a3[pl.ds(off, LANES)]
        for lrep in range(1, LANES):
            r0 = jnp.minimum(r0, a0[pl.ds(lrep * K + off, LANES)])
            r1 = jnp.minimum(r1, a1[pl.ds(lrep * K + off, LANES)])
            r2 = jnp.maximum(r2, a2[pl.ds(lrep * K + off, LANES)])
            r3 = jnp.maximum(r3, a3[pl.ds(lrep * K + off, LANES)])
        part[0, pl.ds(off, LANES)] = r0
        part[1, pl.ds(off, LANES)] = r1
        part[2, pl.ds(off, LANES)] = r2
        part[3, pl.ds(off, LANES)] = r3
        return carry

    lax.fori_loop(0, K // LANES, red_body, 0)

    pltpu.sync_copy(part, shared.at[s])
    plsc.subcore_barrier()

    @pl.when(q == 0)
    def _():
        pltpu.sync_copy(shared.at[pl.ds(s, 4)], comb)

        def comb_body(j, carry):
            off = j * LANES
            for coord, op in ((0, jnp.minimum), (1, jnp.minimum),
                              (2, jnp.maximum), (3, jnp.maximum)):
                r = comb[0, coord, pl.ds(off, LANES)]
                for w in range(1, 4):
                    r = op(r, comb[w, coord, pl.ds(off, LANES)])
                part[coord, pl.ds(off, LANES)] = r
            return carry

        lax.fori_loop(0, K // LANES, comb_body, 0)
        pltpu.sync_copy(part, out_hbm.at[b])


def kernel(boxes, assign_probs, class_logits):
    x1, y1, x2, y2 = _box_call(boxes)
    ha, pc = _tc_call(
        assign_probs.reshape(NTOT, K),
        class_logits.reshape(NTOT, C),
    )
    comp_t = _sc_scatter(ha, x1, y1, x2, y2)
    hard_assign = ha.reshape(B, N)
    pred_classes = pc.reshape(B, N)
    comp = comp_t.transpose(0, 2, 1)
    keep = jnp.ones((B, N), dtype=bool)
    return (hard_assign, pred_classes, boxes, keep, comp)


# speedup vs baseline: 1.3505x; 1.0127x over previous
import functools

import jax
import jax.numpy as jnp
from jax import lax
from jax.experimental import pallas as pl
from jax.experimental.pallas import tpu as pltpu
from jax.experimental.pallas import tpu_sc as plsc

B, N, K, C = 8, 4096, 512, 64
NTOT = B * N
NB = 4096
NTOK = N // 4
LANES = 16


def _tc_body(probs_ref, logits_ref, ha_ref, pc_ref):
    p = probs_ref[...]
    kio = lax.broadcasted_iota(jnp.int32, (NB, K), 1)
    pmax = jnp.max(p, axis=-1, keepdims=True)
    ha_ref[...] = jnp.min(jnp.where(p == pmax, kio, K), axis=-1)

    lt = logits_ref[...].T
    cio = lax.broadcasted_iota(jnp.int32, (C, NB), 0)
    lmax = jnp.max(lt, axis=0, keepdims=True)
    pc_ref[...] = jnp.min(jnp.where(lt == lmax, cio, C), axis=0)


def _tc_call(probs2, logits2):
    tok1 = pl.BlockSpec((NB,), lambda i: (i,))
    out1 = jax.ShapeDtypeStruct((NTOT,), jnp.int32)
    return pl.pallas_call(
        _tc_body,
        grid=(NTOT // NB,),
        in_specs=[
            pl.BlockSpec((NB, K), lambda i: (i, 0)),
            pl.BlockSpec((NB, C), lambda i: (i, 0)),
        ],
        out_specs=[tok1, tok1],
        out_shape=[out1, out1],
        compiler_params=pltpu.CompilerParams(
            dimension_semantics=("arbitrary",),
        ),
    )(probs2, logits2)


def _box_body(boxes_ref, x1_ref, y1_ref, x2_ref, y2_ref):
    bx = boxes_ref[0]
    bt = bx.T
    x1_ref[...] = bt[0]
    y1_ref[...] = bt[1]
    x2_ref[...] = bt[2]
    y2_ref[...] = bt[3]


def _box_call(boxes):
    tok1 = pl.BlockSpec((N,), lambda b: (b,))
    outf = jax.ShapeDtypeStruct((NTOT,), jnp.float32)
    return pl.pallas_call(
        _box_body,
        grid=(B,),
        in_specs=[pl.BlockSpec((1, N, 4), lambda b: (b, 0, 0))],
        out_specs=[tok1, tok1, tok1, tok1],
        out_shape=[outf, outf, outf, outf],
        compiler_params=pltpu.CompilerParams(
            dimension_semantics=("arbitrary",),
        ),
    )(boxes)


@functools.partial(
    pl.kernel,
    mesh=plsc.VectorSubcoreMesh(core_axis_name="c", subcore_axis_name="s"),
    out_type=jax.ShapeDtypeStruct((B, 4, K), jnp.float32),
    scratch_types=[
        pltpu.VMEM((NTOK,), jnp.int32),
        pltpu.VMEM((NTOK,), jnp.float32),
        pltpu.VMEM((NTOK,), jnp.float32),
        pltpu.VMEM((NTOK,), jnp.float32),
        pltpu.VMEM((NTOK,), jnp.float32),
        pltpu.VMEM((LANES * K,), jnp.float32),
        pltpu.VMEM((LANES * K,), jnp.float32),
        pltpu.VMEM((LANES * K,), jnp.float32),
        pltpu.VMEM((LANES * K,), jnp.float32),
        pltpu.VMEM((4, K), jnp.float32),
        pltpu.VMEM((4, 4, K), jnp.float32),
        pltpu.VMEM_SHARED((16, 4, K), jnp.float32),
    ],
    compiler_params=pltpu.CompilerParams(needs_layout_passes=False),
)
def _sc_scatter(ha_hbm, x1_hbm, y1_hbm, x2_hbm, y2_hbm, out_hbm,
                idxv, bx0, bx1, bx2, bx3,
                a0, a1, a2, a3, part, comb, shared):
    c = lax.axis_index("c")
    s = lax.axis_index("s")
    b = c * 4 + s // 4
    q = s % 4
    base = b * N + q * NTOK
    pltpu.sync_copy(ha_hbm.at[pl.ds(base, NTOK)], idxv)
    pltpu.sync_copy(x1_hbm.at[pl.ds(base, NTOK)], bx0)
    pltpu.sync_copy(y1_hbm.at[pl.ds(base, NTOK)], bx1)
    pltpu.sync_copy(x2_hbm.at[pl.ds(base, NTOK)], bx2)
    pltpu.sync_copy(y2_hbm.at[pl.ds(base, NTOK)], bx3)

    ones = jnp.full((LANES,), 1.0, jnp.float32)
    zeros = jnp.zeros((LANES,), jnp.float32)

    def init_body(j, carry):
        off = j * LANES
        a0[pl.ds(off, LANES)] = ones
        a1[pl.ds(off, LANES)] = ones
        a2[pl.ds(off, LANES)] = zeros
        a3[pl.ds(off, LANES)] = zeros
        return carry

    lax.fori_loop(0, K, init_body, 0)

    lane = lax.iota(jnp.int32, LANES) * K

    def tok_body(t, carry):
        off = t * LANES
        g = lane + idxv[pl.ds(off, LANES)]
        v0 = plsc.load_gather(a0, [g])
        plsc.store_scatter(a0, [g], jnp.minimum(v0, bx0[pl.ds(off, LANES)]))
        v1 = plsc.load_gather(a1, [g])
        plsc.store_scatter(a1, [g], jnp.minimum(v1, bx1[pl.ds(off, LANES)]))
        v2 = plsc.load_gather(a2, [g])
        plsc.store_scatter(a2, [g], jnp.maximum(v2, bx2[pl.ds(off, LANES)]))
        v3 = plsc.load_gather(a3, [g])
        plsc.store_scatter(a3, [g], jnp.maximum(v3, bx3[pl.ds(off, LANES)]))
        return carry

    lax.fori_loop(0, NTOK // LANES, tok_body, 0)

    def red_body(j, carry):
        off = j * LANES
        r0 = a0[pl.ds(off, LANES)]
        r1 = a1[pl.ds(off, LANES)]
        r2 = a2[pl.ds(off, LANES)]
        r3 = a3[pl.ds(off, LANES)]
        for lrep in range(1, LANES):
            r0 = jnp.minimum(r0, a0[pl.ds(lrep * K + off, LANES)])
            r1 = jnp.minimum(r1, a1[pl.ds(lrep * K + off, LANES)])
            r2 = jnp.maximum(r2, a2[pl.ds(lrep * K + off, LANES)])
            r3 = jnp.maximum(r3, a3[pl.ds(lrep * K + off, LANES)])
        part[0, pl.ds(off, LANES)] = r0
        part[1, pl.ds(off, LANES)] = r1
        part[2, pl.ds(off, LANES)] = r2
        part[3, pl.ds(off, LANES)] = r3
        return carry

    lax.fori_loop(0, K // LANES, red_body, 0)

    pltpu.sync_copy(part, shared.at[s])
    plsc.subcore_barrier()

    @pl.when(q == 0)
    def _():
        pltpu.sync_copy(shared.at[pl.ds(s, 4)], comb)

        def comb_body(j, carry):
            off = j * LANES
            for coord, op in ((0, jnp.minimum), (1, jnp.minimum),
                              (2, jnp.maximum), (3, jnp.maximum)):
                r = comb[0, coord, pl.ds(off, LANES)]
                for w in range(1, 4):
                    r = op(r, comb[w, coord, pl.ds(off, LANES)])
                part[coord, pl.ds(off, LANES)] = r
            return carry

        lax.fori_loop(0, K // LANES, comb_body, 0)
        pltpu.sync_copy(part, out_hbm.at[b])


def kernel(boxes, assign_probs, class_logits):
    x1, y1, x2, y2 = _box_call(boxes)
    ha, pc = _tc_call(
        assign_probs.reshape(NTOT, K),
        class_logits.reshape(NTOT, C),
    )
    comp_t = _sc_scatter(ha, x1, y1, x2, y2)
    hard_assign = ha.reshape(B, N)
    pred_classes = pc.reshape(B, N)
    comp = comp_t.transpose(0, 2, 1)
    keep = jnp.ones((B, N), dtype=bool)
    return (hard_assign, pred_classes, boxes, keep, comp)
